# baseline (speedup 1.0000x reference)
import functools

import jax
import jax.numpy as jnp
from jax.experimental import pallas as pl
from jax.experimental.pallas import tpu as pltpu


def _ffn_kernel(x_ref, w1_ref, b1_ref, w2_ref, b2_ref, out_ref, p_ref):
    j = pl.program_id(1)
    nj = pl.num_programs(1)
    h = jnp.dot(x_ref[...], w1_ref[...], preferred_element_type=jnp.float32)
    h = jnp.maximum(h + b1_ref[...], 0.0).astype(w2_ref.dtype)
    partial = jnp.dot(h, w2_ref[...], preferred_element_type=jnp.float32)

    b2b = jnp.broadcast_to(b2_ref[...], out_ref.shape)
    prev = jnp.where(j == 0, b2b, p_ref[...])
    acc = jnp.where(j == 0, prev, out_ref[...] + prev)
    out_ref[...] = acc
    p_ref[...] = partial

    @pl.when(j == nj - 1)
    def _():
        out_ref[...] = acc + partial


@functools.partial(jax.jit, static_argnames=("blk_m", "blk_ff"))
def _ffn(x, W1, b1, W2, b2, blk_m=1024, blk_ff=1024):
    n_tok, d_model = x.shape
    d_ff = W1.shape[1]
    blk_m = min(blk_m, n_tok)
    blk_ff = min(blk_ff, d_ff)
    grid = (n_tok // blk_m, d_ff // blk_ff)
    return pl.pallas_call(
        _ffn_kernel,
        grid=grid,
        in_specs=[
            pl.BlockSpec((blk_m, d_model), lambda i, j: (i, 0)),
            pl.BlockSpec((d_model, blk_ff), lambda i, j: (0, j)),
            pl.BlockSpec((blk_ff,), lambda i, j: (j,)),
            pl.BlockSpec((blk_ff, d_model), lambda i, j: (j, 0)),
            pl.BlockSpec((d_model,), lambda i, j: (0,)),
        ],
        out_specs=pl.BlockSpec((blk_m, d_model), lambda i, j: (i, 0)),
        out_shape=jax.ShapeDtypeStruct((n_tok, d_model), jnp.float32),
        scratch_shapes=[pltpu.VMEM((blk_m, d_model), jnp.float32)],
        compiler_params=pltpu.CompilerParams(
            dimension_semantics=("parallel", "arbitrary"),
            vmem_limit_bytes=65472 * 1024,
        ),
    )(x, W1, b1, W2, b2)


def kernel(x, W1, b1, W2, b2):
    bf16 = jnp.bfloat16
    return _ffn(x.astype(bf16), W1.astype(bf16), b1, W2.astype(bf16), b2)


# speedup vs baseline: 1.0247x; 1.0247x over previous
import functools

import jax
import jax.numpy as jnp
from jax.experimental import pallas as pl
from jax.experimental.pallas import tpu as pltpu


def _ffn_kernel(x_ref, w1_ref, b1_ref, w2_ref, b2_ref, out_ref):
    j = pl.program_id(1)
    ff = w1_ref.shape[1]
    half = ff // 2
    parts = []
    for sl in (slice(0, half), slice(half, ff)):
        h = jnp.dot(x_ref[...], w1_ref[:, sl], preferred_element_type=jnp.float32)
        h = jnp.maximum(h + b1_ref[sl], 0.0).astype(w2_ref.dtype)
        parts.append(jnp.dot(h, w2_ref[sl, :], preferred_element_type=jnp.float32))
    partial = parts[0] + parts[1]

    @pl.when(j == 0)
    def _():
        out_ref[...] = partial + b2_ref[...]

    @pl.when(j != 0)
    def _():
        out_ref[...] += partial


@functools.partial(jax.jit, static_argnames=("blk_m", "blk_ff"))
def _ffn(x, W1, b1, W2, b2, blk_m=1024, blk_ff=1024):
    n_tok, d_model = x.shape
    d_ff = W1.shape[1]
    blk_m = min(blk_m, n_tok)
    blk_ff = min(blk_ff, d_ff)
    grid = (n_tok // blk_m, d_ff // blk_ff)
    return pl.pallas_call(
        _ffn_kernel,
        grid=grid,
        in_specs=[
            pl.BlockSpec((blk_m, d_model), lambda i, j: (i, 0)),
            pl.BlockSpec((d_model, blk_ff), lambda i, j: (0, j)),
            pl.BlockSpec((blk_ff,), lambda i, j: (j,)),
            pl.BlockSpec((blk_ff, d_model), lambda i, j: (j, 0)),
            pl.BlockSpec((d_model,), lambda i, j: (0,)),
        ],
        out_specs=pl.BlockSpec((blk_m, d_model), lambda i, j: (i, 0)),
        out_shape=jax.ShapeDtypeStruct((n_tok, d_model), jnp.float32),
        compiler_params=pltpu.CompilerParams(
            dimension_semantics=("parallel", "arbitrary"),
            vmem_limit_bytes=65472 * 1024,
        ),
    )(x, W1, b1, W2, b2)


def kernel(x, W1, b1, W2, b2):
    bf16 = jnp.bfloat16
    return _ffn(x.astype(bf16), W1.astype(bf16), b1, W2.astype(bf16), b2)


# speedup vs baseline: 1.0851x; 1.0590x over previous
import functools

import jax
import jax.numpy as jnp
from jax.experimental import pallas as pl
from jax.experimental.pallas import tpu as pltpu

_VMEM_LIMIT = 65472 * 1024


def _gemm1_kernel(x_ref, w1_ref, b1_ref, h_ref):
    t = jnp.dot(x_ref[...], w1_ref[...], preferred_element_type=jnp.float32)
    h_ref[...] = jnp.maximum(t + b1_ref[...], 0.0).astype(jnp.bfloat16)


def _gemm2_kernel(h_ref, w2_ref, b2_ref, out_ref):
    t = jnp.dot(h_ref[...], w2_ref[...], preferred_element_type=jnp.float32)
    out_ref[...] = t + b2_ref[...]


@functools.partial(jax.jit, static_argnames=("blk_m1", "blk_ff", "blk_m2", "blk_n"))
def _ffn(x, W1, b1, W2, b2, blk_m1=1024, blk_ff=2048, blk_m2=1024, blk_n=512):
    n_tok, d_model = x.shape
    d_ff = W1.shape[1]
    h = pl.pallas_call(
        _gemm1_kernel,
        grid=(n_tok // blk_m1, d_ff // blk_ff),
        in_specs=[
            pl.BlockSpec((blk_m1, d_model), lambda i, j: (i, 0)),
            pl.BlockSpec((d_model, blk_ff), lambda i, j: (0, j)),
            pl.BlockSpec((blk_ff,), lambda i, j: (j,)),
        ],
        out_specs=pl.BlockSpec((blk_m1, blk_ff), lambda i, j: (i, j)),
        out_shape=jax.ShapeDtypeStruct((n_tok, d_ff), jnp.bfloat16),
        compiler_params=pltpu.CompilerParams(
            dimension_semantics=("parallel", "parallel"),
            vmem_limit_bytes=_VMEM_LIMIT,
        ),
    )(x, W1, b1)
    return pl.pallas_call(
        _gemm2_kernel,
        grid=(n_tok // blk_m2, d_model // blk_n),
        in_specs=[
            pl.BlockSpec((blk_m2, d_ff), lambda i, j: (i, 0)),
            pl.BlockSpec((d_ff, blk_n), lambda i, j: (0, j)),
            pl.BlockSpec((blk_n,), lambda i, j: (j,)),
        ],
        out_specs=pl.BlockSpec((blk_m2, blk_n), lambda i, j: (i, j)),
        out_shape=jax.ShapeDtypeStruct((n_tok, d_model), jnp.float32),
        compiler_params=pltpu.CompilerParams(
            dimension_semantics=("parallel", "parallel"),
            vmem_limit_bytes=_VMEM_LIMIT,
        ),
    )(h, W2, b2)


def kernel(x, W1, b1, W2, b2):
    bf16 = jnp.bfloat16
    return _ffn(x.astype(bf16), W1.astype(bf16), b1, W2.astype(bf16), b2)


# speedup vs baseline: 1.1367x; 1.0476x over previous
import functools

import jax
import jax.numpy as jnp
from jax.experimental import pallas as pl
from jax.experimental.pallas import tpu as pltpu

_VMEM_LIMIT = 65472 * 1024


def _gemm1_kernel(x_ref, w1_ref, b1_ref, h_ref):
    t = jnp.dot(x_ref[...], w1_ref[...], preferred_element_type=jnp.float32)
    h_ref[...] = jnp.maximum(t + b1_ref[...], 0.0).astype(jnp.bfloat16)


def _gemm2_kernel(h_ref, w2_ref, b2_ref, out_ref):
    t = jnp.dot(h_ref[...], w2_ref[...], preferred_element_type=jnp.float32)
    out_ref[...] = t + b2_ref[...]


@functools.partial(jax.jit, static_argnames=("blk_m1", "blk_ff", "blk_m2", "blk_n"))
def _ffn(x, W1, b1, W2, b2, blk_m1=1024, blk_ff=2048, blk_m2=1024, blk_n=512):
    n_tok, d_model = x.shape
    d_ff = W1.shape[1]
    h = pl.pallas_call(
        _gemm1_kernel,
        grid=(n_tok // blk_m1, d_ff // blk_ff),
        in_specs=[
            pl.BlockSpec((blk_m1, d_model), lambda i, j: (i, 0)),
            pl.BlockSpec((d_model, blk_ff), lambda i, j: (0, j)),
            pl.BlockSpec((blk_ff,), lambda i, j: (j,)),
        ],
        out_specs=pl.BlockSpec((blk_m1, blk_ff), lambda i, j: (i, j)),
        out_shape=jax.ShapeDtypeStruct((n_tok, d_ff), jnp.bfloat16),
        compiler_params=pltpu.CompilerParams(
            dimension_semantics=("parallel", "parallel"),
            vmem_limit_bytes=_VMEM_LIMIT,
        ),
    )(x, W1, b1)
    return pl.pallas_call(
        _gemm2_kernel,
        grid=(n_tok // blk_m2, d_model // blk_n),
        in_specs=[
            pl.BlockSpec((blk_m2, d_ff), lambda i, j: (i, 0)),
            pl.BlockSpec((d_ff, blk_n), lambda i, j: (0, j)),
            pl.BlockSpec((blk_n,), lambda i, j: (j,)),
        ],
        out_specs=pl.BlockSpec((blk_m2, blk_n), lambda i, j: (i, j)),
        out_shape=jax.ShapeDtypeStruct((n_tok, d_model), jnp.float32),
        compiler_params=pltpu.CompilerParams(
            dimension_semantics=("parallel", "parallel"),
            vmem_limit_bytes=_VMEM_LIMIT,
        ),
    )(h, W2, b2)


def kernel(x, W1, b1, W2, b2):
    bf16 = jnp.bfloat16
    return _ffn(x, W1.astype(bf16), b1, W2.astype(bf16), b2)


# speedup vs baseline: 1.1388x; 1.0018x over previous
import functools

import jax
import jax.numpy as jnp
from jax.experimental import pallas as pl
from jax.experimental.pallas import tpu as pltpu

_VMEM_LIMIT = 65472 * 1024


def _gemm1_kernel(x_ref, w1_ref, b1_ref, h_ref):
    t = jnp.dot(x_ref[...], w1_ref[...], preferred_element_type=jnp.float32)
    h_ref[...] = jnp.maximum(t + b1_ref[...], 0.0).astype(jnp.bfloat16)


def _gemm2_kernel(h_ref, w2_ref, b2_ref, out_ref):
    t = jnp.dot(h_ref[...], w2_ref[...], preferred_element_type=jnp.float32)
    out_ref[...] = t + b2_ref[...]


@functools.partial(jax.jit, static_argnames=("blk_m1", "blk_ff", "blk_m2", "blk_n"))
def _ffn(x, W1, b1, W2, b2, blk_m1=1024, blk_ff=2048, blk_m2=512, blk_n=1024):
    n_tok, d_model = x.shape
    d_ff = W1.shape[1]
    h = pl.pallas_call(
        _gemm1_kernel,
        grid=(n_tok // blk_m1, d_ff // blk_ff),
        in_specs=[
            pl.BlockSpec((blk_m1, d_model), lambda i, j: (i, 0)),
            pl.BlockSpec((d_model, blk_ff), lambda i, j: (0, j)),
            pl.BlockSpec((blk_ff,), lambda i, j: (j,)),
        ],
        out_specs=pl.BlockSpec((blk_m1, blk_ff), lambda i, j: (i, j)),
        out_shape=jax.ShapeDtypeStruct((n_tok, d_ff), jnp.bfloat16),
        compiler_params=pltpu.CompilerParams(
            dimension_semantics=("parallel", "parallel"),
            vmem_limit_bytes=_VMEM_LIMIT,
        ),
    )(x, W1, b1)
    return pl.pallas_call(
        _gemm2_kernel,
        grid=(d_model // blk_n, n_tok // blk_m2),
        in_specs=[
            pl.BlockSpec((blk_m2, d_ff), lambda j, i: (i, 0)),
            pl.BlockSpec((d_ff, blk_n), lambda j, i: (0, j)),
            pl.BlockSpec((blk_n,), lambda j, i: (j,)),
        ],
        out_specs=pl.BlockSpec((blk_m2, blk_n), lambda j, i: (i, j)),
        out_shape=jax.ShapeDtypeStruct((n_tok, d_model), jnp.float32),
        compiler_params=pltpu.CompilerParams(
            dimension_semantics=("parallel", "parallel"),
            vmem_limit_bytes=_VMEM_LIMIT,
        ),
    )(h, W2, b2)


def kernel(x, W1, b1, W2, b2):
    bf16 = jnp.bfloat16
    return _ffn(x, W1.astype(bf16), b1, W2.astype(bf16), b2)
